# Initial kernel scaffold; baseline (speedup 1.0000x reference)
import jax
import jax.numpy as jnp
from jax.experimental import pallas as pl


def kernel(users_emb, items_emb, edge_index, edge_weight):
    raise NotImplementedError("write your pallas kernel here")


# speedup vs baseline: 4.6869x; 4.6869x over previous
import functools

import jax
import jax.numpy as jnp
from jax import lax
from jax.experimental import pallas as pl
from jax.experimental.pallas import tpu as pltpu
from jax.experimental.pallas import tpu_sc as plsc

N_USERS = 25000
N_ITEMS = 25000
NN = N_USERS + N_ITEMS
D = 64
E = 800000

NC = 2
NS = 16
HALF = NN // NC
EPT = E // NS
CH = 400
NCHUNK = EPT // CH
G = 5
SUB = CH // G
ZSTR = 1568
SPREAD = NS * ZSTR
SPREAD_N = 256
ACC_ROWS = SPREAD + SPREAD_N
WSTR = 1560
WTAIL = HALF - NS * WSTR


def _zero_rows(rows):
    z = jnp.zeros((16,), jnp.float32)

    @pl.loop(0, CH)
    def _(i):
        for j in range(D // 16):
            rows[i, pl.ds(j * 16, 16)] = z


def _edge_pass(x_hbm, dst_hbm, src_hbm, w_hbm, acc, dstv, srcv, wv, lidxv,
               rows, sem, base, s):
    ebase = s * EPT

    @pl.loop(0, NCHUNK)
    def _(k):
        off = ebase + k * CH
        pltpu.sync_copy(dst_hbm.at[pl.ds(off, CH)], dstv)
        pltpu.sync_copy(w_hbm.at[pl.ds(off, CH)], wv)
        for g in range(G):
            pltpu.sync_copy(src_hbm.at[pl.ds(off + g * SUB, SUB)], srcv.at[g])

        cps = [
            pltpu.async_copy(x_hbm.at[srcv.at[g]],
                             rows.at[pl.ds(g * SUB, SUB)], sem)
            for g in range(G)
        ]

        @pl.loop(0, G)
        def _(g):
            for j in range(SUB // 16):
                sl = pl.ds(g * SUB + j * 16, 16)
                d16 = dstv[sl]
                w16 = wv[sl]
                m = (d16 >= base) & (d16 < base + HALF)
                lidx = jnp.where(m, d16 - base, SPREAD + (d16 & (SPREAD_N - 1)))
                wv[sl] = jnp.where(m, w16, 0.0)
                lidxv[g, pl.ds(j * 16, 16)] = lidx

        for cp in cps:
            cp.wait()

        @pl.loop(0, CH)
        def _(e):
            wsp = plsc.load_gather(wv, [jnp.broadcast_to(e, (16,))])
            for j in range(D // 16):
                sl = pl.ds(j * 16, 16)
                rows[e, sl] = rows[e, sl] * wsp

        for g in range(G):
            pltpu.sync_copy(rows.at[pl.ds(g * SUB, SUB)],
                            acc.at[lidxv.at[g]], add=True)


def _spmm_body(x_hbm, dst_hbm, src_hbm, w_hbm, y_hbm, acc, dstv, srcv, wv,
               lidxv, rows, sem):
    c = lax.axis_index("c")
    s = lax.axis_index("s")
    base = c * HALF

    _zero_rows(rows)
    for o, sz in ((0, CH), (CH, CH), (2 * CH, CH), (3 * CH, ZSTR - 3 * CH)):
        pltpu.sync_copy(rows.at[pl.ds(0, sz)], acc.at[pl.ds(s * ZSTR + o, sz)])
    plsc.subcore_barrier()

    _edge_pass(x_hbm, dst_hbm, src_hbm, w_hbm, acc, dstv, srcv, wv, lidxv,
               rows, sem, base, s)
    plsc.subcore_barrier()

    def writeback(o, sz):
        pltpu.sync_copy(acc.at[pl.ds(s * WSTR + o, sz)], rows.at[pl.ds(0, sz)])
        pltpu.sync_copy(rows.at[pl.ds(0, sz)],
                        y_hbm.at[pl.ds(base + s * WSTR + o, sz)])

    for o, sz in ((0, CH), (CH, CH), (2 * CH, CH), (3 * CH, WSTR - 3 * CH)):
        writeback(o, sz)

    @pl.when(s == NS - 1)
    def _():
        pltpu.sync_copy(acc.at[pl.ds(NS * WSTR, WTAIL)],
                        rows.at[pl.ds(0, WTAIL)])
        pltpu.sync_copy(rows.at[pl.ds(0, WTAIL)],
                        y_hbm.at[pl.ds(base + NS * WSTR, WTAIL)])


def _final_body(x0_hbm, e1_hbm, dst_hbm, src_hbm, w_hbm,
                mean_hbm, h0_hbm, h1_hbm,
                acc, dstv, srcv, wv, lidxv, rows, sem):
    c = lax.axis_index("c")
    s = lax.axis_index("s")
    base = c * HALF

    _zero_rows(rows)
    for o, sz in ((0, CH), (CH, CH), (2 * CH, CH), (3 * CH, ZSTR - 3 * CH)):
        pltpu.sync_copy(rows.at[pl.ds(0, sz)], acc.at[pl.ds(s * ZSTR + o, sz)])
    plsc.subcore_barrier()

    _edge_pass(e1_hbm, dst_hbm, src_hbm, w_hbm, acc, dstv, srcv, wv, lidxv,
               rows, sem, base, s)
    plsc.subcore_barrier()

    HC = CH // 2
    third = jnp.float32(1.0 / 3.0)

    def combine(o, sz):
        lrow = s * WSTR + o
        grow = base + lrow
        pltpu.sync_copy(acc.at[pl.ds(lrow, sz)], rows.at[pl.ds(0, sz)])
        pltpu.sync_copy(rows.at[pl.ds(0, sz), pl.ds(0, 32)],
                        h0_hbm.at[pl.ds(grow, sz)])
        pltpu.sync_copy(rows.at[pl.ds(0, sz), pl.ds(32, 32)],
                        h1_hbm.at[pl.ds(grow, sz)])
        pltpu.sync_copy(x0_hbm.at[pl.ds(grow, sz)], rows.at[pl.ds(HC, sz)])

        @pl.loop(0, sz)
        def _(r):
            for j in range(D // 16):
                sl = pl.ds(j * 16, 16)
                rows[HC + r, sl] = rows[HC + r, sl] + rows[r, sl]

        pltpu.sync_copy(e1_hbm.at[pl.ds(grow, sz)], rows.at[pl.ds(0, sz)])

        @pl.loop(0, sz)
        def _(r):
            for j in range(D // 16):
                sl = pl.ds(j * 16, 16)
                rows[HC + r, sl] = (rows[HC + r, sl] + rows[r, sl]) * third

        pltpu.sync_copy(rows.at[pl.ds(HC, sz)],
                        mean_hbm.at[pl.ds(grow, sz)])

    for o in range(0, 1400, HC):
        combine(o, HC)
    combine(1400, WSTR - 1400)

    @pl.when(s == NS - 1)
    def _():
        combine(WSTR, WTAIL)


_MESH = plsc.VectorSubcoreMesh(core_axis_name="c", subcore_axis_name="s")

_SCRATCH_COMMON = (
    pltpu.VMEM_SHARED((ACC_ROWS, D), jnp.float32),
    pltpu.VMEM((CH,), jnp.int32),
    pltpu.VMEM((G, SUB), jnp.int32),
    pltpu.VMEM((CH,), jnp.float32),
    pltpu.VMEM((G, SUB), jnp.int32),
    pltpu.VMEM((CH, D), jnp.float32),
)


@functools.partial(
    pl.kernel,
    out_type=jax.ShapeDtypeStruct((NN, D), jnp.float32),
    mesh=_MESH,
    scratch_types=[*_SCRATCH_COMMON, pltpu.SemaphoreType.DMA],
    compiler_params=pltpu.CompilerParams(needs_layout_passes=False, use_tc_tiling_on_sc=False),
)
def _spmm_layer(x_hbm, dst_hbm, src_hbm, w_hbm, y_hbm, *scratch):
    _spmm_body(x_hbm, dst_hbm, src_hbm, w_hbm, y_hbm, *scratch)


@functools.partial(
    pl.kernel,
    out_type=(
        jax.ShapeDtypeStruct((NN, D), jnp.float32),
        jax.ShapeDtypeStruct((NN, D // 2), jnp.float32),
        jax.ShapeDtypeStruct((NN, D // 2), jnp.float32),
    ),
    mesh=_MESH,
    scratch_types=[*_SCRATCH_COMMON, pltpu.SemaphoreType.DMA],
    compiler_params=pltpu.CompilerParams(needs_layout_passes=False, use_tc_tiling_on_sc=False),
)
def _final_layer(x0_hbm, e1_hbm, dst_hbm, src_hbm, w_hbm,
                 mean_hbm, h0_hbm, h1_hbm, *scratch):
    _final_body(x0_hbm, e1_hbm, dst_hbm, src_hbm, w_hbm,
                mean_hbm, h0_hbm, h1_hbm, *scratch)


def kernel(users_emb, items_emb, edge_index, edge_weight):
    x0 = jnp.concatenate([users_emb, items_emb], axis=0)
    dst = edge_index[0]
    src = edge_index[1]
    e1 = _spmm_layer(x0, dst, src, edge_weight)
    mean, h0, h1 = _final_layer(x0, e1, dst, src, edge_weight)
    return (mean[:N_USERS], mean[N_USERS:], h0, h1)


# speedup vs baseline: 10.6946x; 2.2818x over previous
import functools

import jax
import jax.numpy as jnp
from jax import lax
from jax.experimental import pallas as pl
from jax.experimental.pallas import tpu as pltpu
from jax.experimental.pallas import tpu_sc as plsc

N_USERS = 25000
N_ITEMS = 25000
NN = N_USERS + N_ITEMS
D = 64
HD = D // 2
E = 800000

NC = 2
NS = 16
EPT = E // NS
CH = 400
NCHUNK = EPT // CH
G = 5
SUB = CH // G
ACC_ROWS = 50176
ZSTR = ACC_ROWS // NS
WSTR = 3120
WTAIL = NN - NS * WSTR


def _zero_rows(rows):
    z = jnp.zeros((16,), jnp.float32)

    @pl.loop(0, CH)
    def _(i):
        for j in range(HD // 16):
            rows[i, pl.ds(j * 16, 16)] = z


def _gathers(x_hbm, srcf, rows, sem):
    return [
        pltpu.async_copy(x_hbm.at[srcf.at[pl.ds(g * SUB, SUB)]],
                         rows.at[pl.ds(g * SUB, SUB)], sem)
        for g in range(G)
    ]


def _scatters(acc, dst2, rows, sem):
    return [
        pltpu.async_copy(rows.at[pl.ds(g * SUB, SUB)],
                         acc.at[dst2.at[g]], sem, add=True)
        for g in range(G)
    ]


def _sw_loads(src_hbm, w_hbm, off, srcf, wf, sem):
    return [pltpu.async_copy(src_hbm.at[pl.ds(off, CH)], srcf, sem),
            pltpu.async_copy(w_hbm.at[pl.ds(off, CH)], wf, sem)]


def _wait_sw_loads(src_hbm, w_hbm, srcf, wf, sem):
    pltpu.make_async_copy(src_hbm.at[pl.ds(0, CH)], srcf, sem).wait()
    pltpu.make_async_copy(w_hbm.at[pl.ds(0, CH)], wf, sem).wait()


def _dst_loads(dst_hbm, off, dst2, sem):
    return [
        pltpu.async_copy(dst_hbm.at[pl.ds(off + g * SUB, SUB)], dst2.at[g], sem)
        for g in range(G)
    ]


def _wait_dst_loads(dst_hbm, dst2, sem):
    for g in range(G):
        pltpu.make_async_copy(dst_hbm.at[pl.ds(0, SUB)], dst2.at[g],
                              sem).wait()


def _cond(pred, fn):
    if isinstance(pred, bool):
        if pred:
            fn()
    else:
        pl.when(pred)(fn)


def _edge_pass(x_hbm, dst_hbm, src_hbm, w_hbm, acc,
               dst2, srcf, wf, rows, isems, dsems, gsems, ssems, tbase, s):
    ebase = s * EPT

    def adjust_src(b):
        @pl.loop(0, CH // 16)
        def _(i):
            sl = pl.ds(i * 16, 16)
            srcf[b][sl] = srcf[b][sl] + tbase

    def wait_scatters(b):
        for g in range(G):
            pltpu.make_async_copy(rows.at[b].at[pl.ds(g * SUB, SUB)],
                                  acc.at[dst2[b].at[g]], ssems[b]).wait()

    def do_chunk(k, b, stage, waitprev, more):
        nb = 1 - b
        rows_b = rows.at[b]

        for g in range(G):
            pltpu.make_async_copy(x_hbm.at[srcf[b].at[pl.ds(g * SUB, SUB)]],
                                  rows_b.at[pl.ds(g * SUB, SUB)],
                                  gsems[b]).wait()

        if stage:
            _cond(waitprev, lambda: wait_scatters(nb))
            _wait_sw_loads(src_hbm, w_hbm, srcf[nb], wf[nb], isems[nb])
            _dst_loads(dst_hbm, ebase + (k + 1) * CH, dst2[nb], dsems[nb])
            adjust_src(nb)
            _gathers(x_hbm, srcf[nb], rows.at[nb], gsems[nb])

        @pl.loop(0, CH, unroll=4)
        def _(e):
            wsp = plsc.load_gather(wf[b], [jnp.broadcast_to(e, (16,))])
            for j in range(HD // 16):
                sl = pl.ds(j * 16, 16)
                rows_b[e, sl] = rows_b[e, sl] * wsp

        _cond(more, lambda: [
            None for _ in _sw_loads(src_hbm, w_hbm, ebase + (k + 2) * CH,
                                    srcf[b], wf[b], isems[b])])

        _wait_dst_loads(dst_hbm, dst2[b], dsems[b])
        _scatters(acc, dst2[b], rows_b, ssems[b])

    for cp in _sw_loads(src_hbm, w_hbm, ebase, srcf[0], wf[0], isems[0]):
        cp.wait()
    adjust_src(0)
    _dst_loads(dst_hbm, ebase, dst2[0], dsems[0])
    _gathers(x_hbm, srcf[0], rows.at[0], gsems[0])
    _sw_loads(src_hbm, w_hbm, ebase + CH, srcf[1], wf[1], isems[1])

    do_chunk(0, 0, stage=True, waitprev=False, more=True)

    @pl.loop(1, NCHUNK - 2, step=2)
    def _(k):
        do_chunk(k, 1, stage=True, waitprev=True, more=True)
        do_chunk(k + 1, 0, stage=True, waitprev=True, more=True)

    do_chunk(NCHUNK - 2, 1, stage=True, waitprev=True, more=False)
    do_chunk(NCHUNK - 1, 0, stage=False, waitprev=True, more=False)

    wait_scatters(1)
    wait_scatters(0)


def _zero_acc(acc, rows, s):
    _zero_rows(rows.at[0])
    for o in range(0, 2800, CH):
        pltpu.sync_copy(rows.at[0], acc.at[pl.ds(s * ZSTR + o, CH)])
    pltpu.sync_copy(rows.at[0].at[pl.ds(0, ZSTR - 2800)],
                    acc.at[pl.ds(s * ZSTR + 2800, ZSTR - 2800)])


_WCHUNKS = tuple((o, CH) for o in range(0, 2800, CH)) + ((2800, WSTR - 2800),)


def _spmm_body(x_hbm, dst_hbm, src_hbm, w_hbm, y_hbm, acc,
               dst2a, dst2b, srcfa, srcfb, wfa, wfb, rows,
               isem0, isem1, dsem0, dsem1, gsem0, gsem1, ssem0, ssem1):
    c = lax.axis_index("c")
    s = lax.axis_index("s")
    tbase = c * NN

    _zero_acc(acc, rows, s)
    plsc.subcore_barrier()
    _edge_pass(x_hbm, dst_hbm, src_hbm, w_hbm, acc,
               (dst2a, dst2b), (srcfa, srcfb), (wfa, wfb), rows,
               (isem0, isem1), (dsem0, dsem1), (gsem0, gsem1),
               (ssem0, ssem1), tbase, s)
    plsc.subcore_barrier()

    def writeback(o, sz):
        pltpu.sync_copy(acc.at[pl.ds(s * WSTR + o, sz)],
                        rows.at[0].at[pl.ds(0, sz)])
        pltpu.sync_copy(rows.at[0].at[pl.ds(0, sz)],
                        y_hbm.at[pl.ds(tbase + s * WSTR + o, sz)])

    for o, sz in _WCHUNKS:
        writeback(o, sz)

    @pl.when(s == NS - 1)
    def _():
        pltpu.sync_copy(acc.at[pl.ds(NS * WSTR, WTAIL)],
                        rows.at[0].at[pl.ds(0, WTAIL)])
        pltpu.sync_copy(rows.at[0].at[pl.ds(0, WTAIL)],
                        y_hbm.at[pl.ds(tbase + NS * WSTR, WTAIL)])


def _final_body(x0_hbm, e1_hbm, dst_hbm, src_hbm, w_hbm, e2_hbm, mean_hbm,
                acc, dst2a, dst2b, srcfa, srcfb, wfa, wfb, rows,
                isem0, isem1, dsem0, dsem1, gsem0, gsem1, ssem0, ssem1):
    c = lax.axis_index("c")
    s = lax.axis_index("s")
    tbase = c * NN

    _zero_acc(acc, rows, s)
    plsc.subcore_barrier()
    _edge_pass(e1_hbm, dst_hbm, src_hbm, w_hbm, acc,
               (dst2a, dst2b), (srcfa, srcfb), (wfa, wfb), rows,
               (isem0, isem1), (dsem0, dsem1), (gsem0, gsem1),
               (ssem0, ssem1), tbase, s)
    plsc.subcore_barrier()

    third = jnp.float32(1.0 / 3.0)

    def combine(o, sz):
        lrow = s * WSTR + o
        grow = tbase + lrow
        r0 = rows.at[0]
        r1 = rows.at[1]
        pltpu.sync_copy(acc.at[pl.ds(lrow, sz)], r0.at[pl.ds(0, sz)])
        pltpu.sync_copy(r0.at[pl.ds(0, sz)], e2_hbm.at[pl.ds(grow, sz)])
        pltpu.sync_copy(x0_hbm.at[pl.ds(grow, sz)], r1.at[pl.ds(0, sz)])

        @pl.loop(0, sz)
        def _(r):
            for j in range(HD // 16):
                sl = pl.ds(j * 16, 16)
                r1[r, sl] = r1[r, sl] + r0[r, sl]

        pltpu.sync_copy(e1_hbm.at[pl.ds(grow, sz)], r0.at[pl.ds(0, sz)])

        @pl.loop(0, sz)
        def _(r):
            for j in range(HD // 16):
                sl = pl.ds(j * 16, 16)
                r1[r, sl] = (r1[r, sl] + r0[r, sl]) * third

        pltpu.sync_copy(r1.at[pl.ds(0, sz)], mean_hbm.at[pl.ds(grow, sz)])

    for o, sz in _WCHUNKS:
        combine(o, sz)

    @pl.when(s == NS - 1)
    def _():
        combine(WSTR, WTAIL)


_MESH = plsc.VectorSubcoreMesh(core_axis_name="c", subcore_axis_name="s")

_SCRATCH = (
    pltpu.VMEM_SHARED((ACC_ROWS, HD), jnp.float32),
    pltpu.VMEM((G, SUB), jnp.int32),
    pltpu.VMEM((G, SUB), jnp.int32),
    pltpu.VMEM((CH,), jnp.int32),
    pltpu.VMEM((CH,), jnp.int32),
    pltpu.VMEM((CH,), jnp.float32),
    pltpu.VMEM((CH,), jnp.float32),
    pltpu.VMEM((2, CH, HD), jnp.float32),
    pltpu.SemaphoreType.DMA,
    pltpu.SemaphoreType.DMA,
    pltpu.SemaphoreType.DMA,
    pltpu.SemaphoreType.DMA,
    pltpu.SemaphoreType.DMA,
    pltpu.SemaphoreType.DMA,
    pltpu.SemaphoreType.DMA,
    pltpu.SemaphoreType.DMA,
)

_PARAMS = pltpu.CompilerParams(needs_layout_passes=False,
                               use_tc_tiling_on_sc=False)


@functools.partial(
    pl.kernel,
    out_type=jax.ShapeDtypeStruct((NC * NN, HD), jnp.float32),
    mesh=_MESH,
    scratch_types=list(_SCRATCH),
    compiler_params=_PARAMS,
)
def _spmm_layer(x_hbm, dst_hbm, src_hbm, w_hbm, y_hbm, *scratch):
    _spmm_body(x_hbm, dst_hbm, src_hbm, w_hbm, y_hbm, *scratch)


@functools.partial(
    pl.kernel,
    out_type=(
        jax.ShapeDtypeStruct((NC * NN, HD), jnp.float32),
        jax.ShapeDtypeStruct((NC * NN, HD), jnp.float32),
    ),
    mesh=_MESH,
    scratch_types=list(_SCRATCH),
    compiler_params=_PARAMS,
)
def _final_layer(x0_hbm, e1_hbm, dst_hbm, src_hbm, w_hbm, e2_hbm, mean_hbm,
                 *scratch):
    _final_body(x0_hbm, e1_hbm, dst_hbm, src_hbm, w_hbm, e2_hbm, mean_hbm,
                *scratch)


def kernel(users_emb, items_emb, edge_index, edge_weight):
    x0 = jnp.concatenate([users_emb, items_emb], axis=0)
    xs = jnp.concatenate([x0[:, :HD], x0[:, HD:]], axis=0)
    dst = edge_index[0]
    src = edge_index[1]
    e1s = _spmm_layer(xs, dst, src, edge_weight)
    e2s, means = _final_layer(xs, e1s, dst, src, edge_weight)
    light = jnp.concatenate([means[:NN], means[NN:]], axis=1)
    return (light[:N_USERS], light[N_USERS:], e2s[:NN], e2s[NN:])


# speedup vs baseline: 10.7348x; 1.0038x over previous
import functools

import jax
import jax.numpy as jnp
from jax import lax
from jax.experimental import pallas as pl
from jax.experimental.pallas import tpu as pltpu
from jax.experimental.pallas import tpu_sc as plsc

N_USERS = 25000
N_ITEMS = 25000
NN = N_USERS + N_ITEMS
D = 64
HD = D // 2
E = 800000

NC = 2
NS = 16
EPT = E // NS
CH = 400
NCHUNK = EPT // CH
G = 5
SUB = CH // G
ACC_ROWS = 50176
ZSTR = ACC_ROWS // NS
WSTR = 3120
WTAIL = NN - NS * WSTR


def _zero_rows(rows):
    z = jnp.zeros((16,), jnp.float32)

    @pl.loop(0, CH)
    def _(i):
        for j in range(HD // 16):
            rows[i, pl.ds(j * 16, 16)] = z


def _gathers(x_hbm, srcf, rows, sem):
    return [
        pltpu.async_copy(x_hbm.at[srcf.at[pl.ds(g * SUB, SUB)]],
                         rows.at[pl.ds(g * SUB, SUB)], sem)
        for g in range(G)
    ]


def _scatters(acc, dst2, rows, sem):
    return [
        pltpu.async_copy(rows.at[pl.ds(g * SUB, SUB)],
                         acc.at[dst2.at[g]], sem, add=True)
        for g in range(G)
    ]


def _sw_loads(src_hbm, w_hbm, off, srcf, wf, sem):
    return [pltpu.async_copy(src_hbm.at[pl.ds(off, CH)], srcf, sem),
            pltpu.async_copy(w_hbm.at[pl.ds(off, CH)], wf, sem)]


def _wait_sw_loads(src_hbm, w_hbm, srcf, wf, sem):
    pltpu.make_async_copy(src_hbm.at[pl.ds(0, CH)], srcf, sem).wait()
    pltpu.make_async_copy(w_hbm.at[pl.ds(0, CH)], wf, sem).wait()


def _dst_loads(dst_hbm, off, dst2, sem):
    return [
        pltpu.async_copy(dst_hbm.at[pl.ds(off + g * SUB, SUB)], dst2.at[g], sem)
        for g in range(G)
    ]


def _wait_dst_loads(dst_hbm, dst2, sem):
    for g in range(G):
        pltpu.make_async_copy(dst_hbm.at[pl.ds(0, SUB)], dst2.at[g],
                              sem).wait()


def _cond(pred, fn):
    if isinstance(pred, bool):
        if pred:
            fn()
    else:
        pl.when(pred)(fn)


def _edge_pass(x_hbm, dst_hbm, src_hbm, w_hbm, acc,
               dst2, srcf, wf, rows, isems, dsems, gsems, ssems, tbase, s):
    ebase = s * EPT

    def adjust_src(b):
        @pl.loop(0, CH // 16)
        def _(i):
            sl = pl.ds(i * 16, 16)
            srcf[b][sl] = srcf[b][sl] + tbase

    def wait_scatters(b):
        for g in range(G):
            pltpu.make_async_copy(rows.at[b].at[pl.ds(g * SUB, SUB)],
                                  acc.at[dst2[b].at[g]], ssems[b]).wait()

    def do_chunk(k, b, stage, waitprev, more):
        nb = 1 - b
        rows_b = rows.at[b]

        for g in range(G):
            pltpu.make_async_copy(x_hbm.at[srcf[b].at[pl.ds(g * SUB, SUB)]],
                                  rows_b.at[pl.ds(g * SUB, SUB)],
                                  gsems[b]).wait()

        if stage:
            _cond(waitprev, lambda: wait_scatters(nb))
            _wait_sw_loads(src_hbm, w_hbm, srcf[nb], wf[nb], isems[nb])
            _dst_loads(dst_hbm, ebase + (k + 1) * CH, dst2[nb], dsems[nb])
            adjust_src(nb)
            _gathers(x_hbm, srcf[nb], rows.at[nb], gsems[nb])

        @pl.loop(0, CH, unroll=4)
        def _(e):
            wsp = plsc.load_gather(wf[b], [jnp.broadcast_to(e, (16,))])
            for j in range(HD // 16):
                sl = pl.ds(j * 16, 16)
                rows_b[e, sl] = rows_b[e, sl] * wsp

        _cond(more, lambda: [
            None for _ in _sw_loads(src_hbm, w_hbm, ebase + (k + 2) * CH,
                                    srcf[b], wf[b], isems[b])])

        _wait_dst_loads(dst_hbm, dst2[b], dsems[b])
        _scatters(acc, dst2[b], rows_b, ssems[b])

    for cp in _sw_loads(src_hbm, w_hbm, ebase, srcf[0], wf[0], isems[0]):
        cp.wait()
    adjust_src(0)
    _dst_loads(dst_hbm, ebase, dst2[0], dsems[0])
    _gathers(x_hbm, srcf[0], rows.at[0], gsems[0])
    _sw_loads(src_hbm, w_hbm, ebase + CH, srcf[1], wf[1], isems[1])

    do_chunk(0, 0, stage=True, waitprev=False, more=True)

    @pl.loop(1, NCHUNK - 2, step=2)
    def _(k):
        do_chunk(k, 1, stage=True, waitprev=True, more=True)
        do_chunk(k + 1, 0, stage=True, waitprev=True, more=True)

    do_chunk(NCHUNK - 2, 1, stage=True, waitprev=True, more=False)
    do_chunk(NCHUNK - 1, 0, stage=False, waitprev=True, more=False)

    wait_scatters(1)
    wait_scatters(0)


def _zero_acc(acc, rows, s):
    _zero_rows(rows.at[0])
    for o in range(0, 2800, CH):
        pltpu.sync_copy(rows.at[0], acc.at[pl.ds(s * ZSTR + o, CH)])
    pltpu.sync_copy(rows.at[0].at[pl.ds(0, ZSTR - 2800)],
                    acc.at[pl.ds(s * ZSTR + 2800, ZSTR - 2800)])


_WCHUNKS = tuple((o, CH) for o in range(0, 2800, CH)) + ((2800, WSTR - 2800),)


def _fused_body(x_hbm, dst_hbm, src_hbm, w_hbm, e1_hbm, e2_hbm, mean_hbm,
                acc, dst2a, dst2b, srcfa, srcfb, wfa, wfb, rows,
                isem0, isem1, dsem0, dsem1, gsem0, gsem1, ssem0, ssem1):
    c = lax.axis_index("c")
    s = lax.axis_index("s")
    tbase = c * NN

    sems = ((isem0, isem1), (dsem0, dsem1), (gsem0, gsem1), (ssem0, ssem1))
    bufs = ((dst2a, dst2b), (srcfa, srcfb), (wfa, wfb), rows)

    _zero_acc(acc, rows, s)
    plsc.subcore_barrier()
    _edge_pass(x_hbm, dst_hbm, src_hbm, w_hbm, acc, *bufs, *sems, tbase, s)
    plsc.subcore_barrier()

    def writeback(o, sz):
        pltpu.sync_copy(acc.at[pl.ds(s * WSTR + o, sz)],
                        rows.at[0].at[pl.ds(0, sz)])
        pltpu.sync_copy(rows.at[0].at[pl.ds(0, sz)],
                        e1_hbm.at[pl.ds(tbase + s * WSTR + o, sz)])

    for o, sz in _WCHUNKS:
        writeback(o, sz)

    @pl.when(s == NS - 1)
    def _():
        pltpu.sync_copy(acc.at[pl.ds(NS * WSTR, WTAIL)],
                        rows.at[0].at[pl.ds(0, WTAIL)])
        pltpu.sync_copy(rows.at[0].at[pl.ds(0, WTAIL)],
                        e1_hbm.at[pl.ds(tbase + NS * WSTR, WTAIL)])

    plsc.subcore_barrier()
    _zero_acc(acc, rows, s)
    plsc.subcore_barrier()

    _edge_pass(e1_hbm, dst_hbm, src_hbm, w_hbm, acc, *bufs, *sems, tbase, s)
    plsc.subcore_barrier()

    third = jnp.float32(1.0 / 3.0)

    def combine(o, sz):
        lrow = s * WSTR + o
        grow = tbase + lrow
        r0 = rows.at[0]
        r1 = rows.at[1]
        pltpu.sync_copy(acc.at[pl.ds(lrow, sz)], r0.at[pl.ds(0, sz)])
        pltpu.sync_copy(r0.at[pl.ds(0, sz)], e2_hbm.at[pl.ds(grow, sz)])
        pltpu.sync_copy(x_hbm.at[pl.ds(grow, sz)], r1.at[pl.ds(0, sz)])

        @pl.loop(0, sz)
        def _(r):
            for j in range(HD // 16):
                sl = pl.ds(j * 16, 16)
                r1[r, sl] = r1[r, sl] + r0[r, sl]

        pltpu.sync_copy(e1_hbm.at[pl.ds(grow, sz)], r0.at[pl.ds(0, sz)])

        @pl.loop(0, sz)
        def _(r):
            for j in range(HD // 16):
                sl = pl.ds(j * 16, 16)
                r1[r, sl] = (r1[r, sl] + r0[r, sl]) * third

        pltpu.sync_copy(r1.at[pl.ds(0, sz)], mean_hbm.at[pl.ds(grow, sz)])

    for o, sz in _WCHUNKS:
        combine(o, sz)

    @pl.when(s == NS - 1)
    def _():
        combine(WSTR, WTAIL)


_MESH = plsc.VectorSubcoreMesh(core_axis_name="c", subcore_axis_name="s")

_SCRATCH = (
    pltpu.VMEM_SHARED((ACC_ROWS, HD), jnp.float32),
    pltpu.VMEM((G, SUB), jnp.int32),
    pltpu.VMEM((G, SUB), jnp.int32),
    pltpu.VMEM((CH,), jnp.int32),
    pltpu.VMEM((CH,), jnp.int32),
    pltpu.VMEM((CH,), jnp.float32),
    pltpu.VMEM((CH,), jnp.float32),
    pltpu.VMEM((2, CH, HD), jnp.float32),
    pltpu.SemaphoreType.DMA,
    pltpu.SemaphoreType.DMA,
    pltpu.SemaphoreType.DMA,
    pltpu.SemaphoreType.DMA,
    pltpu.SemaphoreType.DMA,
    pltpu.SemaphoreType.DMA,
    pltpu.SemaphoreType.DMA,
    pltpu.SemaphoreType.DMA,
)

_PARAMS = pltpu.CompilerParams(needs_layout_passes=False,
                               use_tc_tiling_on_sc=False)


@functools.partial(
    pl.kernel,
    out_type=(
        jax.ShapeDtypeStruct((NC * NN, HD), jnp.float32),
        jax.ShapeDtypeStruct((NC * NN, HD), jnp.float32),
        jax.ShapeDtypeStruct((NC * NN, HD), jnp.float32),
    ),
    mesh=_MESH,
    scratch_types=list(_SCRATCH),
    compiler_params=_PARAMS,
)
def _fused(x_hbm, dst_hbm, src_hbm, w_hbm, e1_hbm, e2_hbm, mean_hbm,
           *scratch):
    _fused_body(x_hbm, dst_hbm, src_hbm, w_hbm, e1_hbm, e2_hbm, mean_hbm,
                *scratch)


def kernel(users_emb, items_emb, edge_index, edge_weight):
    x0 = jnp.concatenate([users_emb, items_emb], axis=0)
    xs = jnp.concatenate([x0[:, :HD], x0[:, HD:]], axis=0)
    dst = edge_index[0]
    src = edge_index[1]
    _, e2s, means = _fused(xs, dst, src, edge_weight)
    light = jnp.concatenate([means[:NN], means[NN:]], axis=1)
    return (light[:N_USERS], light[N_USERS:], e2s[:NN], e2s[NN:])


# speedup vs baseline: 16.7352x; 1.5590x over previous
import functools

import jax
import jax.numpy as jnp
from jax import lax
from jax.experimental import pallas as pl
from jax.experimental.pallas import tpu as pltpu
from jax.experimental.pallas import tpu_sc as plsc

N_USERS = 25000
N_ITEMS = 25000
NN = N_USERS + N_ITEMS
D = 64
HD = D // 2
E = 800000

NC = 2
NS = 16
EPT = E // NS
CH = 400
NCHUNK = EPT // CH
G = 5
SUB = CH // G
ACC_ROWS = 50176
ZSTR = ACC_ROWS // NS
WSTR = 3120
WTAIL = NN - NS * WSTR


def _zero_rows(rows):
    z = jnp.zeros((16,), jnp.float32)

    @plsc.parallel_loop(0, CH, unroll=8)
    def _(i):
        for j in range(HD // 16):
            rows[i, pl.ds(j * 16, 16)] = z


def _gathers(x_hbm, srcf, rows, sem):
    return [
        pltpu.async_copy(x_hbm.at[srcf.at[pl.ds(g * SUB, SUB)]],
                         rows.at[pl.ds(g * SUB, SUB)], sem)
        for g in range(G)
    ]


def _scatters(acc, dst2, rows, sem):
    return [
        pltpu.async_copy(rows.at[pl.ds(g * SUB, SUB)],
                         acc.at[dst2.at[g]], sem, add=True)
        for g in range(G)
    ]


def _sw_loads(src_hbm, w_hbm, off, srcf, wf, sem):
    return [pltpu.async_copy(src_hbm.at[pl.ds(off, CH)], srcf, sem),
            pltpu.async_copy(w_hbm.at[pl.ds(off, CH)], wf, sem)]


def _wait_sw_loads(src_hbm, w_hbm, srcf, wf, sem):
    pltpu.make_async_copy(src_hbm.at[pl.ds(0, CH)], srcf, sem).wait()
    pltpu.make_async_copy(w_hbm.at[pl.ds(0, CH)], wf, sem).wait()


def _dst_loads(dst_hbm, off, dst2, sem):
    return [
        pltpu.async_copy(dst_hbm.at[pl.ds(off + g * SUB, SUB)], dst2.at[g], sem)
        for g in range(G)
    ]


def _wait_dst_loads(dst_hbm, dst2, sem):
    for g in range(G):
        pltpu.make_async_copy(dst_hbm.at[pl.ds(0, SUB)], dst2.at[g],
                              sem).wait()


def _cond(pred, fn):
    if isinstance(pred, bool):
        if pred:
            fn()
    else:
        pl.when(pred)(fn)


def _edge_pass(x_hbm, dst_hbm, src_hbm, w_hbm, acc,
               dst2, srcf, wf, rows, isems, dsems, gsems, ssems, tbase, s):
    ebase = s * EPT

    def adjust_src(b):
        @plsc.parallel_loop(0, CH // 16, unroll=5)
        def _(i):
            sl = pl.ds(i * 16, 16)
            srcf[b][sl] = srcf[b][sl] + tbase

    def wait_scatters(b):
        for g in range(G):
            pltpu.make_async_copy(rows.at[b].at[pl.ds(g * SUB, SUB)],
                                  acc.at[dst2[b].at[g]], ssems[b]).wait()

    def do_chunk(k, b, stage, waitprev, more):
        nb = 1 - b
        rows_b = rows.at[b]

        for g in range(G):
            pltpu.make_async_copy(x_hbm.at[srcf[b].at[pl.ds(g * SUB, SUB)]],
                                  rows_b.at[pl.ds(g * SUB, SUB)],
                                  gsems[b]).wait()

        if stage:
            _cond(waitprev, lambda: wait_scatters(nb))
            _wait_sw_loads(src_hbm, w_hbm, srcf[nb], wf[nb], isems[nb])
            _dst_loads(dst_hbm, ebase + (k + 1) * CH, dst2[nb], dsems[nb])
            adjust_src(nb)
            _gathers(x_hbm, srcf[nb], rows.at[nb], gsems[nb])

        @plsc.parallel_loop(0, CH, unroll=8)
        def _(e):
            wsp = plsc.load_gather(wf[b], [jnp.broadcast_to(e, (16,))])
            for j in range(HD // 16):
                sl = pl.ds(j * 16, 16)
                rows_b[e, sl] = rows_b[e, sl] * wsp

        _cond(more, lambda: [
            None for _ in _sw_loads(src_hbm, w_hbm, ebase + (k + 2) * CH,
                                    srcf[b], wf[b], isems[b])])

        _wait_dst_loads(dst_hbm, dst2[b], dsems[b])
        _scatters(acc, dst2[b], rows_b, ssems[b])

    for cp in _sw_loads(src_hbm, w_hbm, ebase, srcf[0], wf[0], isems[0]):
        cp.wait()
    adjust_src(0)
    _dst_loads(dst_hbm, ebase, dst2[0], dsems[0])
    _gathers(x_hbm, srcf[0], rows.at[0], gsems[0])
    _sw_loads(src_hbm, w_hbm, ebase + CH, srcf[1], wf[1], isems[1])

    do_chunk(0, 0, stage=True, waitprev=False, more=True)

    @pl.loop(1, NCHUNK - 2, step=2)
    def _(k):
        do_chunk(k, 1, stage=True, waitprev=True, more=True)
        do_chunk(k + 1, 0, stage=True, waitprev=True, more=True)

    do_chunk(NCHUNK - 2, 1, stage=True, waitprev=True, more=False)
    do_chunk(NCHUNK - 1, 0, stage=False, waitprev=True, more=False)

    wait_scatters(1)
    wait_scatters(0)


def _zero_acc(acc, rows, s):
    _zero_rows(rows.at[0])
    for o in range(0, 2800, CH):
        pltpu.sync_copy(rows.at[0], acc.at[pl.ds(s * ZSTR + o, CH)])
    pltpu.sync_copy(rows.at[0].at[pl.ds(0, ZSTR - 2800)],
                    acc.at[pl.ds(s * ZSTR + 2800, ZSTR - 2800)])


_WCHUNKS = tuple((o, CH) for o in range(0, 2800, CH)) + ((2800, WSTR - 2800),)


def _fused_body(x_hbm, dst_hbm, src_hbm, w_hbm, e1_hbm, e2_hbm, mean_hbm,
                acc, dst2a, dst2b, srcfa, srcfb, wfa, wfb, rows,
                isem0, isem1, dsem0, dsem1, gsem0, gsem1, ssem0, ssem1):
    c = lax.axis_index("c")
    s = lax.axis_index("s")
    tbase = c * NN

    sems = ((isem0, isem1), (dsem0, dsem1), (gsem0, gsem1), (ssem0, ssem1))
    bufs = ((dst2a, dst2b), (srcfa, srcfb), (wfa, wfb), rows)

    _zero_acc(acc, rows, s)
    plsc.subcore_barrier()
    _edge_pass(x_hbm, dst_hbm, src_hbm, w_hbm, acc, *bufs, *sems, tbase, s)
    plsc.subcore_barrier()

    def writeback(o, sz):
        pltpu.sync_copy(acc.at[pl.ds(s * WSTR + o, sz)],
                        rows.at[0].at[pl.ds(0, sz)])
        pltpu.sync_copy(rows.at[0].at[pl.ds(0, sz)],
                        e1_hbm.at[pl.ds(tbase + s * WSTR + o, sz)])

    for o, sz in _WCHUNKS:
        writeback(o, sz)

    @pl.when(s == NS - 1)
    def _():
        pltpu.sync_copy(acc.at[pl.ds(NS * WSTR, WTAIL)],
                        rows.at[0].at[pl.ds(0, WTAIL)])
        pltpu.sync_copy(rows.at[0].at[pl.ds(0, WTAIL)],
                        e1_hbm.at[pl.ds(tbase + NS * WSTR, WTAIL)])

    plsc.subcore_barrier()
    _zero_acc(acc, rows, s)
    plsc.subcore_barrier()

    _edge_pass(e1_hbm, dst_hbm, src_hbm, w_hbm, acc, *bufs, *sems, tbase, s)
    plsc.subcore_barrier()

    third = jnp.float32(1.0 / 3.0)

    def combine(o, sz):
        lrow = s * WSTR + o
        grow = tbase + lrow
        r0 = rows.at[0]
        r1 = rows.at[1]
        pltpu.sync_copy(acc.at[pl.ds(lrow, sz)], r0.at[pl.ds(0, sz)])
        pltpu.sync_copy(r0.at[pl.ds(0, sz)], e2_hbm.at[pl.ds(grow, sz)])
        pltpu.sync_copy(x_hbm.at[pl.ds(grow, sz)], r1.at[pl.ds(0, sz)])

        @plsc.parallel_loop(0, sz, unroll=8)
        def _(r):
            for j in range(HD // 16):
                sl = pl.ds(j * 16, 16)
                r1[r, sl] = r1[r, sl] + r0[r, sl]

        pltpu.sync_copy(e1_hbm.at[pl.ds(grow, sz)], r0.at[pl.ds(0, sz)])

        @plsc.parallel_loop(0, sz, unroll=8)
        def _(r):
            for j in range(HD // 16):
                sl = pl.ds(j * 16, 16)
                r1[r, sl] = (r1[r, sl] + r0[r, sl]) * third

        pltpu.sync_copy(r1.at[pl.ds(0, sz)], mean_hbm.at[pl.ds(grow, sz)])

    for o, sz in _WCHUNKS:
        combine(o, sz)

    @pl.when(s == NS - 1)
    def _():
        combine(WSTR, WTAIL)


_MESH = plsc.VectorSubcoreMesh(core_axis_name="c", subcore_axis_name="s")

_SCRATCH = (
    pltpu.VMEM_SHARED((ACC_ROWS, HD), jnp.float32),
    pltpu.VMEM((G, SUB), jnp.int32),
    pltpu.VMEM((G, SUB), jnp.int32),
    pltpu.VMEM((CH,), jnp.int32),
    pltpu.VMEM((CH,), jnp.int32),
    pltpu.VMEM((CH,), jnp.float32),
    pltpu.VMEM((CH,), jnp.float32),
    pltpu.VMEM((2, CH, HD), jnp.float32),
    pltpu.SemaphoreType.DMA,
    pltpu.SemaphoreType.DMA,
    pltpu.SemaphoreType.DMA,
    pltpu.SemaphoreType.DMA,
    pltpu.SemaphoreType.DMA,
    pltpu.SemaphoreType.DMA,
    pltpu.SemaphoreType.DMA,
    pltpu.SemaphoreType.DMA,
)

_PARAMS = pltpu.CompilerParams(needs_layout_passes=False,
                               use_tc_tiling_on_sc=False)


@functools.partial(
    pl.kernel,
    out_type=(
        jax.ShapeDtypeStruct((NC * NN, HD), jnp.float32),
        jax.ShapeDtypeStruct((NC * NN, HD), jnp.float32),
        jax.ShapeDtypeStruct((NC * NN, HD), jnp.float32),
    ),
    mesh=_MESH,
    scratch_types=list(_SCRATCH),
    compiler_params=_PARAMS,
)
def _fused(x_hbm, dst_hbm, src_hbm, w_hbm, e1_hbm, e2_hbm, mean_hbm,
           *scratch):
    _fused_body(x_hbm, dst_hbm, src_hbm, w_hbm, e1_hbm, e2_hbm, mean_hbm,
                *scratch)


def kernel(users_emb, items_emb, edge_index, edge_weight):
    x0 = jnp.concatenate([users_emb, items_emb], axis=0)
    xs = jnp.concatenate([x0[:, :HD], x0[:, HD:]], axis=0)
    dst = edge_index[0]
    src = edge_index[1]
    _, e2s, means = _fused(xs, dst, src, edge_weight)
    light = jnp.concatenate([means[:NN], means[NN:]], axis=1)
    return (light[:N_USERS], light[N_USERS:], e2s[:NN], e2s[NN:])
